# Initial kernel scaffold; baseline (speedup 1.0000x reference)
import jax
import jax.numpy as jnp
from jax.experimental import pallas as pl


def kernel(dm1, edges_t, dm_t, edges_s, dm_s, edges_g, dm_g, W_t1, b_t1, W_t2, b_t2, W_s1, b_s1, W_s2, b_s2, W_g1, b_g1, W_g2, b_g2, fc1_W, fc1_b, fc2_W, fc2_b, cnn_W, cnn_b):
    raise NotImplementedError("write your pallas kernel here")


# speedup vs baseline: 36.7429x; 36.7429x over previous
import functools

import jax
import jax.numpy as jnp
from jax import lax
from jax.experimental import pallas as pl
from jax.experimental.pallas import tpu as pltpu
from jax.experimental.pallas import tpu_sc as plsc

_N = 591
_FD = 512
_E = 37824
_NN = _N * _N
_PW = 21840
_NNP = 16 * _PW
_CH = 1280
_NWIN = _CH // 128


def _sc_counts(ets, etd, ess, esd, egs, egd):
    mesh = plsc.VectorSubcoreMesh(core_axis_name="c", subcore_axis_name="s")

    @functools.partial(
        pl.kernel,
        out_type=jax.ShapeDtypeStruct((6 * _NNP,), jnp.float32),
        mesh=mesh,
        scratch_types=[
            pltpu.VMEM((_CH,), jnp.int32),
            pltpu.VMEM((_CH,), jnp.int32),
            pltpu.VMEM((_NWIN, 128), jnp.int32),
            pltpu.VMEM((_NWIN, 128), jnp.float32),
            pltpu.VMEM((_PW,), jnp.float32),
            pltpu.VMEM((_PW,), jnp.float32),
            pltpu.VMEM_SHARED((_NNP,), jnp.float32),
        ],
        compiler_params=pltpu.CompilerParams(needs_layout_passes=False),
    )
    def k(ets_hbm, etd_hbm, ess_hbm, esd_hbm, egs_hbm, egd_hbm, out_hbm,
          src_v, dst_v, idx_v, val_v, zb_v, cp_v, cnts):
        cid = lax.axis_index("c")
        sid = lax.axis_index("s")
        wid = sid * 2 + cid
        base = wid * _CH
        dma_base = jnp.minimum(base, _E - _CH)
        lo = base
        hi = jnp.minimum(base + _CH, _E)

        @pl.loop(0, _PW, step=16)
        def _(i):
            zb_v[pl.ds(i, 16)] = jnp.zeros((16,), jnp.float32)

        for br, (s_hbm, d_hbm) in enumerate(
                ((ets_hbm, etd_hbm), (ess_hbm, esd_hbm), (egs_hbm, egd_hbm))):
            pltpu.sync_copy(zb_v, cnts.at[pl.ds(sid * _PW, _PW)])
            plsc.subcore_barrier()

            pltpu.sync_copy(s_hbm.at[pl.ds(dma_base, _CH)], src_v)
            pltpu.sync_copy(d_hbm.at[pl.ds(dma_base, _CH)], dst_v)

            for w in range(_NWIN):
                @pl.loop(0, 128, step=16)
                def _(cc, w=w):
                    j = w * 128 + cc
                    s = src_v[pl.ds(j, 16)]
                    d = dst_v[pl.ds(j, 16)]
                    g = dma_base + j + lax.iota(jnp.int32, 16)
                    m = (g >= lo) & (g < hi)
                    idx_v[w, pl.ds(cc, 16)] = s * _N + d
                    val_v[w, pl.ds(cc, 16)] = m.astype(jnp.float32)

            for w in range(_NWIN):
                pltpu.sync_copy(val_v.at[w], cnts.at[idx_v.at[w]], add=True)
            plsc.subcore_barrier()

            pltpu.sync_copy(cnts.at[pl.ds(sid * _PW, _PW)], cp_v)
            pltpu.sync_copy(
                cp_v,
                out_hbm.at[pl.ds((br * 2 + cid) * _NNP + sid * _PW, _PW)],
            )

    return k(ets, etd, ess, esd, egs, egd)


def _tc_body(x_ref, ct_ref, dmt_ref, cs_ref, dms_ref, cg_ref, dmg_ref,
             wt1, bt1, wt2, bt2, ws1, bs1, ws2, bs2, wg1, bg1, wg2, bg2,
             f1w, f1b, f2w, f2b, cw, cb, out_ref):
    f32 = jnp.float32
    x = x_ref[...]
    ones_col = jnp.ones((_N, 1), f32)
    tdims = (((0,), (0,)), ((), ()))

    hs = []
    for c_ref, dm_ref, W1, b1, W2, b2 in (
            (ct_ref, dmt_ref, wt1, bt1, wt2, bt2),
            (cs_ref, dms_ref, ws1, bs1, ws2, bs2),
            (cg_ref, dmg_ref, wg1, bg1, wg2, bg2)):
        M = (c_ref[0] + c_ref[1]) * dm_ref[...]
        deg_col = lax.dot_general(M, ones_col, tdims,
                                  preferred_element_type=f32) + 1.0
        deg_row = jnp.sum(M, axis=0, keepdims=True) + 1.0
        dis_col = lax.rsqrt(deg_col)
        dis_row = lax.rsqrt(deg_row)
        Mn = M * dis_row * dis_col
        sl = dis_col * dis_col
        h = x
        for W, b in ((W1, b1), (W2, b2)):
            xw = jnp.dot(h, W[...], preferred_element_type=f32)
            agg = lax.dot_general(Mn, xw, tdims, preferred_element_type=f32)
            h = jax.nn.relu(agg + sl * xw + b[...])
            hs.append(h)

    inv = 1.0 / (_N * _FD)
    gap_row = jnp.concatenate(
        [jnp.sum(h) * inv * jnp.ones((1, 1), f32) for h in hs], axis=1)
    a1 = jax.nn.relu(jnp.dot(gap_row, f1w[...], preferred_element_type=f32)
                     + f1b[...])
    a2 = jax.nn.sigmoid(jnp.dot(a1, f2w[...], preferred_element_type=f32)
                        + f2b[...])
    cwv = cw[...]
    out = jnp.zeros((_N, _FD), f32) + cb[...]
    for c, h in enumerate(hs):
        out = out + cwv[0:1, c:c + 1] * jax.nn.relu(a2[0:1, c:c + 1] * h)
    out_ref[...] = out


def kernel(dm1, edges_t, dm_t, edges_s, dm_s, edges_g, dm_g,
           W_t1, b_t1, W_t2, b_t2, W_s1, b_s1, W_s2, b_s2,
           W_g1, b_g1, W_g2, b_g2,
           fc1_W, fc1_b, fc2_W, fc2_b, cnn_W, cnn_b):
    counts = _sc_counts(edges_t[0], edges_t[1], edges_s[0], edges_s[1],
                        edges_g[0], edges_g[1]).reshape(3, 2, _NNP)
    cnt = counts[:, :, :_NN].reshape(3, 2, _N, _N)

    out = pl.pallas_call(
        _tc_body,
        out_shape=jax.ShapeDtypeStruct((_N, _FD), jnp.float32),
        compiler_params=pltpu.CompilerParams(vmem_limit_bytes=100 * 1024 * 1024),
    )(dm1, cnt[0], dm_t, cnt[1], dm_s, cnt[2], dm_g,
      W_t1, b_t1.reshape(1, _FD), W_t2, b_t2.reshape(1, _FD),
      W_s1, b_s1.reshape(1, _FD), W_s2, b_s2.reshape(1, _FD),
      W_g1, b_g1.reshape(1, _FD), W_g2, b_g2.reshape(1, _FD),
      fc1_W, fc1_b.reshape(1, 30), fc2_W, fc2_b.reshape(1, 6),
      cnn_W.reshape(1, 6), cnn_b.reshape(1, 1))
    return out


# speedup vs baseline: 58.8973x; 1.6030x over previous
import functools

import jax
import jax.numpy as jnp
from jax import lax
from jax.experimental import pallas as pl
from jax.experimental.pallas import tpu as pltpu
from jax.experimental.pallas import tpu_sc as plsc

_N = 591
_FD = 512
_E = 37824
_NB = 640
_BLK = _N * _NB
_PW = _BLK // 16
_CH = 1280
_NWIN = _CH // 128


def _sc_counts(et, es, eg):
    mesh = plsc.VectorSubcoreMesh(core_axis_name="c", subcore_axis_name="s")

    @functools.partial(
        pl.kernel,
        out_type=jax.ShapeDtypeStruct((6 * _BLK,), jnp.float32),
        mesh=mesh,
        scratch_types=(
            [pltpu.VMEM((_CH,), jnp.int32)] * 3
            + [pltpu.VMEM((_CH,), jnp.int32)] * 3
            + [pltpu.VMEM((_NWIN, 128), jnp.int32)] * 3
            + [pltpu.VMEM((_NWIN, 128), jnp.float32)] * 3
            + [pltpu.VMEM((_PW,), jnp.float32)]
            + [pltpu.VMEM((_PW,), jnp.float32)] * 2
            + [pltpu.VMEM_SHARED((_BLK,), jnp.float32)]
            + [pltpu.SemaphoreType.DMA] * 3
        ),
        compiler_params=pltpu.CompilerParams(needs_layout_passes=False),
    )
    def k(et_hbm, es_hbm, eg_hbm, out_hbm,
          s0, s1, s2, d0, d1, d2, i0, i1, i2, v0, v1, v2,
          zb_v, p0, p1, cnts, sem_e, sem_s, sem_o):
        src_v = (s0, s1, s2)
        dst_v = (d0, d1, d2)
        idx_v = (i0, i1, i2)
        val_v = (v0, v1, v2)
        cp_v = (p0, p1)
        cid = lax.axis_index("c")
        sid = lax.axis_index("s")
        wid = sid * 2 + cid
        base = wid * _CH
        dma_base = jnp.minimum(base, _E - _CH)
        lo = base
        hi = jnp.minimum(base + _CH, _E)

        ecopies = []
        for br, e_hbm in enumerate((et_hbm, es_hbm, eg_hbm)):
            ecopies.append(pltpu.async_copy(
                e_hbm.at[pl.ds(dma_base, _CH)], src_v[br], sem_e))
            ecopies.append(pltpu.async_copy(
                e_hbm.at[pl.ds(_E + dma_base, _CH)], dst_v[br], sem_e))

        @pl.loop(0, _PW, step=16)
        def _(i):
            zb_v[pl.ds(i, 16)] = jnp.zeros((16,), jnp.float32)

        zcopy = pltpu.async_copy(zb_v, cnts.at[pl.ds(sid * _PW, _PW)], sem_s)

        for c in ecopies:
            c.wait()

        for br in range(3):
            for w in range(_NWIN):
                @pl.loop(0, 128, step=16)
                def _(cc, br=br, w=w):
                    j = w * 128 + cc
                    s = src_v[br][pl.ds(j, 16)]
                    d = dst_v[br][pl.ds(j, 16)]
                    g = dma_base + j + lax.iota(jnp.int32, 16)
                    m = (g >= lo) & (g < hi)
                    idx_v[br][w, pl.ds(cc, 16)] = s * _NB + d
                    val_v[br][w, pl.ds(cc, 16)] = m.astype(jnp.float32)

        zcopy.wait()
        hcopies = []
        for br in range(3):
            plsc.subcore_barrier()
            scopies = []
            for w in range(_NWIN):
                scopies.append(pltpu.async_copy(
                    val_v[br].at[w], cnts.at[idx_v[br].at[w]],
                    sem_s, add=True))
            for c in scopies:
                c.wait()
            plsc.subcore_barrier()

            if br == 2:
                hcopies[0].wait()
            ocopy = pltpu.async_copy(
                cnts.at[pl.ds(sid * _PW, _PW)], cp_v[br % 2], sem_o)
            ocopy.wait()
            off = (br * 2 + cid) * _BLK + sid * _PW
            hcopies.append(pltpu.async_copy(
                cp_v[br % 2], out_hbm.at[pl.ds(off, _PW)], sem_e))
            if br < 2:
                pltpu.async_copy(zb_v, cnts.at[pl.ds(sid * _PW, _PW)],
                                 sem_s).wait()
        for c in hcopies[1:]:
            c.wait()

    return k(et, es, eg)


def _tc_body(x_ref, ct_ref, dmt_ref, cs_ref, dms_ref, cg_ref, dmg_ref,
             wt1, bt1, wt2, bt2, ws1, bs1, ws2, bs2, wg1, bg1, wg2, bg2,
             f1w, f1b, f2w, f2b, cw, cb, out_ref):
    f32 = jnp.float32
    x = x_ref[...]
    ones_col = jnp.ones((_N, 1), f32)
    tdims = (((0,), (0,)), ((), ()))

    hs = []
    for c_ref, dm_ref, W1, b1, W2, b2 in (
            (ct_ref, dmt_ref, wt1, bt1, wt2, bt2),
            (cs_ref, dms_ref, ws1, bs1, ws2, bs2),
            (cg_ref, dmg_ref, wg1, bg1, wg2, bg2)):
        Bc = (c_ref[0] + c_ref[1])[:, :_N]
        M = Bc * dm_ref[...]
        deg_col = lax.dot_general(M, ones_col, tdims,
                                  preferred_element_type=f32) + 1.0
        deg_row = jnp.sum(M, axis=0, keepdims=True) + 1.0
        dis_col = lax.rsqrt(deg_col)
        dis_row = lax.rsqrt(deg_row)
        Mn = M * dis_row * dis_col
        sl = dis_col * dis_col
        h = x
        for W, b in ((W1, b1), (W2, b2)):
            xw = jnp.dot(h, W[...], preferred_element_type=f32)
            agg = lax.dot_general(Mn, xw, tdims, preferred_element_type=f32)
            h = jax.nn.relu(agg + sl * xw + b[...])
            hs.append(h)

    inv = 1.0 / (_N * _FD)
    gap_row = jnp.concatenate(
        [jnp.sum(h) * inv * jnp.ones((1, 1), f32) for h in hs], axis=1)
    a1 = jax.nn.relu(jnp.dot(gap_row, f1w[...], preferred_element_type=f32)
                     + f1b[...])
    a2 = jax.nn.sigmoid(jnp.dot(a1, f2w[...], preferred_element_type=f32)
                        + f2b[...])
    cwv = cw[...]
    out = jnp.zeros((_N, _FD), f32) + cb[...]
    for c, h in enumerate(hs):
        out = out + cwv[0:1, c:c + 1] * jax.nn.relu(a2[0:1, c:c + 1] * h)
    out_ref[...] = out


def kernel(dm1, edges_t, dm_t, edges_s, dm_s, edges_g, dm_g,
           W_t1, b_t1, W_t2, b_t2, W_s1, b_s1, W_s2, b_s2,
           W_g1, b_g1, W_g2, b_g2,
           fc1_W, fc1_b, fc2_W, fc2_b, cnn_W, cnn_b):
    counts = _sc_counts(edges_t.reshape(2 * _E), edges_s.reshape(2 * _E),
                        edges_g.reshape(2 * _E))
    cnt = counts.reshape(3, 2, _N, _NB)

    out = pl.pallas_call(
        _tc_body,
        out_shape=jax.ShapeDtypeStruct((_N, _FD), jnp.float32),
        compiler_params=pltpu.CompilerParams(vmem_limit_bytes=100 * 1024 * 1024),
    )(dm1, cnt[0], dm_t, cnt[1], dm_s, cnt[2], dm_g,
      W_t1, b_t1.reshape(1, _FD), W_t2, b_t2.reshape(1, _FD),
      W_s1, b_s1.reshape(1, _FD), W_s2, b_s2.reshape(1, _FD),
      W_g1, b_g1.reshape(1, _FD), W_g2, b_g2.reshape(1, _FD),
      fc1_W, fc1_b.reshape(1, 30), fc2_W, fc2_b.reshape(1, 6),
      cnn_W.reshape(1, 6), cnn_b.reshape(1, 1))
    return out


# speedup vs baseline: 73.4832x; 1.2476x over previous
import functools

import jax
import jax.numpy as jnp
from jax import lax
from jax.experimental import pallas as pl
from jax.experimental.pallas import tpu as pltpu
from jax.experimental.pallas import tpu_sc as plsc

_N = 591
_FD = 512
_E = 37824
_NB = 640
_BLK = _NB * _NB
_RPS = _NB // 16
_PW = _RPS * _NB
_CH = 1280
_NWIN = _CH // 128


def _sc_counts(et, es, eg):
    mesh = plsc.VectorSubcoreMesh(core_axis_name="c", subcore_axis_name="s")

    @functools.partial(
        pl.kernel,
        out_type=jax.ShapeDtypeStruct((6 * _NB, _NB), jnp.float32),
        mesh=mesh,
        scratch_types=(
            [pltpu.VMEM((_CH,), jnp.int32)] * 3
            + [pltpu.VMEM((_CH,), jnp.int32)] * 3
            + [pltpu.VMEM((_NWIN, 128), jnp.int32)] * 3
            + [pltpu.VMEM((_NWIN, 128), jnp.float32)] * 3
            + [pltpu.VMEM((_PW,), jnp.float32)]
            + [pltpu.VMEM((_RPS, _NB), jnp.float32)] * 2
            + [pltpu.VMEM_SHARED((_BLK,), jnp.float32)]
            + [pltpu.SemaphoreType.DMA] * 3
        ),
        compiler_params=pltpu.CompilerParams(needs_layout_passes=False),
    )
    def k(et_hbm, es_hbm, eg_hbm, out_hbm,
          s0, s1, s2, d0, d1, d2, i0, i1, i2, v0, v1, v2,
          zb_v, p0, p1, cnts, sem_e, sem_s, sem_o):
        src_v = (s0, s1, s2)
        dst_v = (d0, d1, d2)
        idx_v = (i0, i1, i2)
        val_v = (v0, v1, v2)
        cp_v = (p0, p1)
        cid = lax.axis_index("c")
        sid = lax.axis_index("s")
        wid = sid * 2 + cid
        base = wid * _CH
        dma_base = jnp.minimum(base, _E - _CH)
        lo = base
        hi = jnp.minimum(base + _CH, _E)

        ecopies = []
        for br, e_hbm in enumerate((et_hbm, es_hbm, eg_hbm)):
            ecopies.append(pltpu.async_copy(
                e_hbm.at[pl.ds(dma_base, _CH)], src_v[br], sem_e))
            ecopies.append(pltpu.async_copy(
                e_hbm.at[pl.ds(_E + dma_base, _CH)], dst_v[br], sem_e))

        @pl.loop(0, _PW, step=16)
        def _(i):
            zb_v[pl.ds(i, 16)] = jnp.zeros((16,), jnp.float32)

        zcopy = pltpu.async_copy(zb_v, cnts.at[pl.ds(sid * _PW, _PW)], sem_s)

        for c in ecopies:
            c.wait()

        for br in range(3):
            for w in range(_NWIN):
                @pl.loop(0, 128, step=16)
                def _(cc, br=br, w=w):
                    j = w * 128 + cc
                    s = src_v[br][pl.ds(j, 16)]
                    d = dst_v[br][pl.ds(j, 16)]
                    g = dma_base + j + lax.iota(jnp.int32, 16)
                    m = (g >= lo) & (g < hi)
                    idx_v[br][w, pl.ds(cc, 16)] = s * _NB + d
                    val_v[br][w, pl.ds(cc, 16)] = m.astype(jnp.float32)

        zcopy.wait()
        hcopies = []
        for br in range(3):
            plsc.subcore_barrier()
            scopies = []
            for w in range(_NWIN):
                scopies.append(pltpu.async_copy(
                    val_v[br].at[w], cnts.at[idx_v[br].at[w]],
                    sem_s, add=True))
            for c in scopies:
                c.wait()
            plsc.subcore_barrier()

            if br == 2:
                hcopies[0].wait()
            ocopies = [
                pltpu.async_copy(
                    cnts.at[pl.ds(sid * _PW + r * _NB, _NB)],
                    cp_v[br % 2].at[r], sem_o)
                for r in range(_RPS)
            ]
            for c in ocopies:
                c.wait()
            row0 = (br * 2 + cid) * _NB + sid * _RPS
            hcopies.append(pltpu.async_copy(
                cp_v[br % 2], out_hbm.at[pl.ds(row0, _RPS), :], sem_e))
            if br < 2:
                pltpu.async_copy(zb_v, cnts.at[pl.ds(sid * _PW, _PW)],
                                 sem_s).wait()
        for c in hcopies[1:]:
            c.wait()

    return k(et, es, eg)


def _tc_body(x_ref, cnt_ref, dmt_ref, dms_ref, dmg_ref,
             wt1, bt1, wt2, bt2, ws1, bs1, ws2, bs2, wg1, bg1, wg2, bg2,
             f1w, f1b, f2w, f2b, cw, cb, out_ref):
    f32 = jnp.float32
    x = x_ref[...]
    ones_col = jnp.ones((_N, 1), f32)
    tdims = (((0,), (0,)), ((), ()))

    hs = []
    for br, (dm_ref, W1, b1, W2, b2) in enumerate(
            ((dmt_ref, wt1, bt1, wt2, bt2),
             (dms_ref, ws1, bs1, ws2, bs2),
             (dmg_ref, wg1, bg1, wg2, bg2))):
        r0 = 2 * br * _NB
        Bc = (cnt_ref[pl.ds(r0, _N), :_N]
              + cnt_ref[pl.ds(r0 + _NB, _N), :_N])
        M = Bc * dm_ref[...]
        deg_col = lax.dot_general(M, ones_col, tdims,
                                  preferred_element_type=f32) + 1.0
        deg_row = jnp.sum(M, axis=0, keepdims=True) + 1.0
        dis_col = lax.rsqrt(deg_col)
        dis_row = lax.rsqrt(deg_row)
        Mn = M * dis_row * dis_col
        sl = dis_col * dis_col
        h = x
        for W, b in ((W1, b1), (W2, b2)):
            xw = jnp.dot(h, W[...], preferred_element_type=f32)
            agg = lax.dot_general(Mn, xw, tdims, preferred_element_type=f32)
            h = jax.nn.relu(agg + sl * xw + b[...])
            hs.append(h)

    inv = 1.0 / (_N * _FD)
    gap_row = jnp.concatenate(
        [jnp.sum(h) * inv * jnp.ones((1, 1), f32) for h in hs], axis=1)
    a1 = jax.nn.relu(jnp.dot(gap_row, f1w[...], preferred_element_type=f32)
                     + f1b[...])
    a2 = jax.nn.sigmoid(jnp.dot(a1, f2w[...], preferred_element_type=f32)
                        + f2b[...])
    cwv = cw[...]
    out = jnp.zeros((_N, _FD), f32) + cb[...]
    for c, h in enumerate(hs):
        out = out + cwv[0:1, c:c + 1] * jax.nn.relu(a2[0:1, c:c + 1] * h)
    out_ref[...] = out


def kernel(dm1, edges_t, dm_t, edges_s, dm_s, edges_g, dm_g,
           W_t1, b_t1, W_t2, b_t2, W_s1, b_s1, W_s2, b_s2,
           W_g1, b_g1, W_g2, b_g2,
           fc1_W, fc1_b, fc2_W, fc2_b, cnn_W, cnn_b):
    counts = _sc_counts(edges_t.reshape(2 * _E), edges_s.reshape(2 * _E),
                        edges_g.reshape(2 * _E))

    out = pl.pallas_call(
        _tc_body,
        out_shape=jax.ShapeDtypeStruct((_N, _FD), jnp.float32),
        compiler_params=pltpu.CompilerParams(vmem_limit_bytes=100 * 1024 * 1024),
    )(dm1, counts, dm_t, dm_s, dm_g,
      W_t1, b_t1.reshape(1, _FD), W_t2, b_t2.reshape(1, _FD),
      W_s1, b_s1.reshape(1, _FD), W_s2, b_s2.reshape(1, _FD),
      W_g1, b_g1.reshape(1, _FD), W_g2, b_g2.reshape(1, _FD),
      fc1_W, fc1_b.reshape(1, 30), fc2_W, fc2_b.reshape(1, 6),
      cnn_W.reshape(1, 6), cnn_b.reshape(1, 1))
    return out


# speedup vs baseline: 79.5671x; 1.0828x over previous
import functools

import jax
import jax.numpy as jnp
from jax import lax
from jax.experimental import pallas as pl
from jax.experimental.pallas import tpu as pltpu
from jax.experimental.pallas import tpu_sc as plsc

_N = 591
_FD = 512
_E = 37824
_NB = 640
_BLK = _NB * _NB
_RPS = _NB // 16
_PW = _RPS * _NB
_CH = 1280
_NWIN = _CH // 128
_EAL = (_E // 128) * 128
_TC0 = _EAL - 128
_ZCH = _PW // 4


def _sc_counts(et, es, eg):
    mesh = plsc.VectorSubcoreMesh(core_axis_name="c", subcore_axis_name="s")

    @functools.partial(
        pl.kernel,
        out_type=jax.ShapeDtypeStruct((6 * _NB, _NB), jnp.float32),
        mesh=mesh,
        scratch_types=(
            [pltpu.VMEM((_CH,), jnp.int32)] * 6
            + [pltpu.VMEM((128,), jnp.int32)] * 6
            + [pltpu.VMEM((_NWIN, 128), jnp.int32)] * 3
            + [pltpu.VMEM((_NWIN, 128), jnp.float32)] * 3
            + [pltpu.VMEM((_ZCH,), jnp.float32)]
            + [pltpu.VMEM((_RPS, _NB), jnp.float32)] * 2
            + [pltpu.VMEM_SHARED((_BLK,), jnp.float32)] * 2
            + [pltpu.SemaphoreType.DMA] * 3
        ),
        compiler_params=pltpu.CompilerParams(needs_layout_passes=False),
    )
    def k(et_hbm, es_hbm, eg_hbm, out_hbm,
          es0, ed0, es1, ed1, es2, ed2, ts0, td0, ts1, td1, ts2, td2,
          i0, i1, i2, v0, v1, v2,
          zb_v, p0, p1, blk0, blk1, sem_e, sem_s, sem_o):
        esrc_v = (es0, es1, es2)
        edst_v = (ed0, ed1, ed2)
        tsrc_v = (ts0, ts1, ts2)
        tdst_v = (td0, td1, td2)
        idx_v = (i0, i1, i2)
        val_v = (v0, v1, v2)
        cp_v = (p0, p1)
        blks = (blk0, blk1, blk0)
        cid = lax.axis_index("c")
        sid = lax.axis_index("s")
        wid = sid * 2 + cid
        lo = wid * _CH
        hi = jnp.minimum(lo + _CH, _E)
        c0 = jnp.minimum(lo, _E - _CH)

        ecopies = []
        for br, e_hbm in enumerate((et_hbm, es_hbm, eg_hbm)):
            ecopies.append(pltpu.async_copy(
                e_hbm.at[pl.ds(c0, _CH)], esrc_v[br], sem_e))
            ecopies.append(pltpu.async_copy(
                e_hbm.at[pl.ds(_E + c0, _CH)], edst_v[br], sem_e))

        @pl.loop(0, _ZCH, step=16)
        def _(i):
            zb_v[pl.ds(i, 16)] = jnp.zeros((16,), jnp.float32)

        zcopies = [
            pltpu.async_copy(
                zb_v, blk.at[pl.ds(sid * _PW + q * _ZCH, _ZCH)], sem_s)
            for blk in (blk0, blk1) for q in range(4)
        ]

        for c in ecopies:
            c.wait()

        for br in range(3):
            for w in range(_NWIN):
                @pl.loop(0, 128, step=16)
                def _(cc, br=br, w=w):
                    j = w * 128 + cc
                    s = esrc_v[br][pl.ds(j, 16)]
                    d = edst_v[br][pl.ds(j, 16)]
                    g = c0 + j + lax.iota(jnp.int32, 16)
                    m = (g >= lo) & (g < hi)
                    idx_v[br][w, pl.ds(cc, 16)] = s * _NB + d
                    val_v[br][w, pl.ds(cc, 16)] = m.astype(jnp.float32)

        for c in zcopies:
            c.wait()
        plsc.subcore_barrier()

        for br in (0, 1):
            scopies = []
            for w in range(_NWIN):
                scopies.append(pltpu.async_copy(
                    val_v[br].at[w], blks[br].at[idx_v[br].at[w]],
                    sem_s, add=True))
            for c in scopies:
                c.wait()
        plsc.subcore_barrier()

        hcopies = []
        for br in (0, 1):
            ocopies = [
                pltpu.async_copy(
                    blks[br].at[pl.ds(sid * _PW + r * _NB, _NB)],
                    cp_v[br].at[r], sem_o)
                for r in range(_RPS)
            ]
            for c in ocopies:
                c.wait()
            row0 = (br * 2 + cid) * _NB + sid * _RPS
            hcopies.append(pltpu.async_copy(
                cp_v[br], out_hbm.at[pl.ds(row0, _RPS), :], sem_e))
        rz = [
            pltpu.async_copy(
                zb_v, blk0.at[pl.ds(sid * _PW + q * _ZCH, _ZCH)], sem_s)
            for q in range(4)
        ]
        for c in rz:
            c.wait()
        plsc.subcore_barrier()

        scopies = []
        for w in range(_NWIN):
            scopies.append(pltpu.async_copy(
                val_v[2].at[w], blk0.at[idx_v[2].at[w]], sem_s, add=True))
        for c in scopies:
            c.wait()
        plsc.subcore_barrier()

        hcopies[0].wait()
        ocopies = [
            pltpu.async_copy(
                blk0.at[pl.ds(sid * _PW + r * _NB, _NB)],
                cp_v[0].at[r], sem_o)
            for r in range(_RPS)
        ]
        for c in ocopies:
            c.wait()
        row0 = (2 * 2 + cid) * _NB + sid * _RPS
        hlast = pltpu.async_copy(
            cp_v[0], out_hbm.at[pl.ds(row0, _RPS), :], sem_e)
        hcopies[1].wait()
        hlast.wait()

    return k(et, es, eg)


def _tc_body(x_ref, cnt_ref, dmt_ref, dms_ref, dmg_ref,
             wt1, bt1, wt2, bt2, ws1, bs1, ws2, bs2, wg1, bg1, wg2, bg2,
             f1w, f1b, f2w, f2b, cw, cb, out_ref):
    f32 = jnp.float32
    x = x_ref[...]
    ones_col = jnp.ones((_N, 1), f32)
    tdims = (((0,), (0,)), ((), ()))

    hs = []
    for br, (dm_ref, W1, b1, W2, b2) in enumerate(
            ((dmt_ref, wt1, bt1, wt2, bt2),
             (dms_ref, ws1, bs1, ws2, bs2),
             (dmg_ref, wg1, bg1, wg2, bg2))):
        r0 = 2 * br * _NB
        Bc = (cnt_ref[pl.ds(r0, _N), :_N]
              + cnt_ref[pl.ds(r0 + _NB, _N), :_N])
        M = Bc * dm_ref[...]
        deg_col = lax.dot_general(M, ones_col, tdims,
                                  preferred_element_type=f32) + 1.0
        deg_row = jnp.sum(M, axis=0, keepdims=True) + 1.0
        dis_col = lax.rsqrt(deg_col)
        dis_row = lax.rsqrt(deg_row)
        Mn = M * dis_row * dis_col
        sl = dis_col * dis_col
        h = x
        for W, b in ((W1, b1), (W2, b2)):
            xw = jnp.dot(h, W[...], preferred_element_type=f32)
            agg = lax.dot_general(Mn, xw, tdims, preferred_element_type=f32)
            h = jax.nn.relu(agg + sl * xw + b[...])
            hs.append(h)

    inv = 1.0 / (_N * _FD)
    gap_row = jnp.concatenate(
        [jnp.sum(h) * inv * jnp.ones((1, 1), f32) for h in hs], axis=1)
    a1 = jax.nn.relu(jnp.dot(gap_row, f1w[...], preferred_element_type=f32)
                     + f1b[...])
    a2 = jax.nn.sigmoid(jnp.dot(a1, f2w[...], preferred_element_type=f32)
                        + f2b[...])
    cwv = cw[...]
    out = jnp.zeros((_N, _FD), f32) + cb[...]
    for c, h in enumerate(hs):
        out = out + cwv[0:1, c:c + 1] * jax.nn.relu(a2[0:1, c:c + 1] * h)
    out_ref[...] = out


def kernel(dm1, edges_t, dm_t, edges_s, dm_s, edges_g, dm_g,
           W_t1, b_t1, W_t2, b_t2, W_s1, b_s1, W_s2, b_s2,
           W_g1, b_g1, W_g2, b_g2,
           fc1_W, fc1_b, fc2_W, fc2_b, cnn_W, cnn_b):
    counts = _sc_counts(edges_t.reshape(2 * _E), edges_s.reshape(2 * _E),
                        edges_g.reshape(2 * _E))

    out = pl.pallas_call(
        _tc_body,
        out_shape=jax.ShapeDtypeStruct((_N, _FD), jnp.float32),
        compiler_params=pltpu.CompilerParams(vmem_limit_bytes=100 * 1024 * 1024),
    )(dm1, counts, dm_t, dm_s, dm_g,
      W_t1, b_t1.reshape(1, _FD), W_t2, b_t2.reshape(1, _FD),
      W_s1, b_s1.reshape(1, _FD), W_s2, b_s2.reshape(1, _FD),
      W_g1, b_g1.reshape(1, _FD), W_g2, b_g2.reshape(1, _FD),
      fc1_W, fc1_b.reshape(1, 30), fc2_W, fc2_b.reshape(1, 6),
      cnn_W.reshape(1, 6), cnn_b.reshape(1, 1))
    return out


# speedup vs baseline: 79.8296x; 1.0033x over previous
import functools

import jax
import jax.numpy as jnp
from jax import lax
from jax.experimental import pallas as pl
from jax.experimental.pallas import tpu as pltpu
from jax.experimental.pallas import tpu_sc as plsc

_N = 591
_FD = 512
_E = 37824
_NB = 640
_BLK = _NB * _NB
_RPS = _NB // 16
_PW = _RPS * _NB
_CH = 1280
_NWIN = _CH // 128
_EAL = (_E // 128) * 128
_TC0 = _EAL - 128
_ZCH = _PW // 4


def _sc_counts(et, es, eg):
    mesh = plsc.VectorSubcoreMesh(core_axis_name="c", subcore_axis_name="s")

    @functools.partial(
        pl.kernel,
        out_type=jax.ShapeDtypeStruct((6 * _NB, _NB), jnp.float32),
        mesh=mesh,
        scratch_types=(
            [pltpu.VMEM((_CH,), jnp.int32)] * 6
            + [pltpu.VMEM((128,), jnp.int32)] * 6
            + [pltpu.VMEM((_NWIN, 128), jnp.int32)] * 3
            + [pltpu.VMEM((_NWIN, 128), jnp.float32)] * 3
            + [pltpu.VMEM((_ZCH,), jnp.float32)]
            + [pltpu.VMEM((_RPS, _NB), jnp.float32)] * 2
            + [pltpu.VMEM_SHARED((_BLK,), jnp.float32)] * 2
            + [pltpu.SemaphoreType.DMA] * 3
        ),
        compiler_params=pltpu.CompilerParams(needs_layout_passes=False),
    )
    def k(et_hbm, es_hbm, eg_hbm, out_hbm,
          es0, ed0, es1, ed1, es2, ed2, ts0, td0, ts1, td1, ts2, td2,
          i0, i1, i2, v0, v1, v2,
          zb_v, p0, p1, blk0, blk1, sem_e, sem_s, sem_o):
        esrc_v = (es0, es1, es2)
        edst_v = (ed0, ed1, ed2)
        tsrc_v = (ts0, ts1, ts2)
        tdst_v = (td0, td1, td2)
        idx_v = (i0, i1, i2)
        val_v = (v0, v1, v2)
        cp_v = (p0, p1)
        blks = (blk0, blk1, blk0)
        cid = lax.axis_index("c")
        sid = lax.axis_index("s")
        wid = sid * 2 + cid
        lo = wid * _CH
        hi = jnp.minimum(lo + _CH, _E)
        c0 = jnp.minimum(lo, _E - _CH)

        ecopies = []
        for br, e_hbm in enumerate((et_hbm, es_hbm, eg_hbm)):
            ecopies.append(pltpu.async_copy(
                e_hbm.at[pl.ds(c0, _CH)], esrc_v[br], sem_e))
            ecopies.append(pltpu.async_copy(
                e_hbm.at[pl.ds(_E + c0, _CH)], edst_v[br], sem_e))

        @pl.loop(0, _ZCH, step=16)
        def _(i):
            zb_v[pl.ds(i, 16)] = jnp.zeros((16,), jnp.float32)

        zcopies = [
            pltpu.async_copy(
                zb_v, blk.at[pl.ds(sid * _PW + q * _ZCH, _ZCH)], sem_s)
            for blk in (blk0, blk1) for q in range(4)
        ]

        for c in ecopies:
            c.wait()

        for br in range(3):
            for w in range(_NWIN):
                @pl.loop(0, 128, step=16)
                def _(cc, br=br, w=w):
                    j = w * 128 + cc
                    s = esrc_v[br][pl.ds(j, 16)]
                    d = edst_v[br][pl.ds(j, 16)]
                    g = c0 + j + lax.iota(jnp.int32, 16)
                    m = (g >= lo) & (g < hi)
                    idx_v[br][w, pl.ds(cc, 16)] = s * _NB + d
                    val_v[br][w, pl.ds(cc, 16)] = m.astype(jnp.float32)

        for c in zcopies:
            c.wait()
        plsc.subcore_barrier()

        for br in (0, 1):
            scopies = []
            for w in range(_NWIN):
                scopies.append(pltpu.async_copy(
                    val_v[br].at[w], blks[br].at[idx_v[br].at[w]],
                    sem_s, add=True))
            for c in scopies:
                c.wait()
        plsc.subcore_barrier()

        hcopies = []
        for br in (0, 1):
            ocopies = [
                pltpu.async_copy(
                    blks[br].at[pl.ds(sid * _PW + r * _NB, _NB)],
                    cp_v[br].at[r], sem_o)
                for r in range(_RPS)
            ]
            for c in ocopies:
                c.wait()
            row0 = (br * 2 + cid) * _NB + sid * _RPS
            hcopies.append(pltpu.async_copy(
                cp_v[br], out_hbm.at[pl.ds(row0, _RPS), :], sem_e))
        rz = [
            pltpu.async_copy(
                zb_v, blk0.at[pl.ds(sid * _PW + q * _ZCH, _ZCH)], sem_s)
            for q in range(4)
        ]
        for c in rz:
            c.wait()
        plsc.subcore_barrier()

        scopies = []
        for w in range(_NWIN):
            scopies.append(pltpu.async_copy(
                val_v[2].at[w], blk0.at[idx_v[2].at[w]], sem_s, add=True))
        for c in scopies:
            c.wait()
        plsc.subcore_barrier()

        hcopies[0].wait()
        ocopies = [
            pltpu.async_copy(
                blk0.at[pl.ds(sid * _PW + r * _NB, _NB)],
                cp_v[0].at[r], sem_o)
            for r in range(_RPS)
        ]
        for c in ocopies:
            c.wait()
        row0 = (2 * 2 + cid) * _NB + sid * _RPS
        hlast = pltpu.async_copy(
            cp_v[0], out_hbm.at[pl.ds(row0, _RPS), :], sem_e)
        hcopies[1].wait()
        hlast.wait()

    return k(et, es, eg)


def _tc_body(x_ref, cnt_ref, dm_ref, w1_ref, b1_ref, w2_ref, b2_ref,
             f1w, f1b, f2w, f2b, cw, cb, out_ref, hs_ref):
    f32 = jnp.float32
    br = pl.program_id(0)
    x = x_ref[...]
    ones_col = jnp.ones((_N, 1), f32)
    tdims = (((0,), (0,)), ((), ()))

    Bc = (cnt_ref[pl.ds(0, _N), :_N]
          + cnt_ref[pl.ds(_NB, _N), :_N])
    M = Bc * dm_ref[0]
    deg_col = lax.dot_general(M, ones_col, tdims,
                              preferred_element_type=f32) + 1.0
    deg_row = jnp.sum(M, axis=0, keepdims=True) + 1.0
    dis_col = lax.rsqrt(deg_col)
    dis_row = lax.rsqrt(deg_row)
    Mn = M * dis_row * dis_col
    sl = dis_col * dis_col
    h = x
    for li, (W, b) in enumerate(((w1_ref, b1_ref), (w2_ref, b2_ref))):
        xw = jnp.dot(h, W[0], preferred_element_type=f32)
        agg = lax.dot_general(Mn, xw, tdims, preferred_element_type=f32)
        h = jax.nn.relu(agg + sl * xw + b[0])
        hs_ref[2 * br + li] = h

    @pl.when(br == 2)
    def _():
        inv = 1.0 / (_N * _FD)
        hs = [hs_ref[c] for c in range(6)]
        gap_row = jnp.concatenate(
            [jnp.sum(hc) * inv * jnp.ones((1, 1), f32) for hc in hs],
            axis=1)
        a1 = jax.nn.relu(jnp.dot(gap_row, f1w[...],
                                 preferred_element_type=f32) + f1b[...])
        a2 = jax.nn.sigmoid(jnp.dot(a1, f2w[...],
                                    preferred_element_type=f32) + f2b[...])
        cwv = cw[...]
        out = jnp.zeros((_N, _FD), f32) + cb[...]
        for c, hc in enumerate(hs):
            out = out + cwv[0:1, c:c + 1] * jax.nn.relu(a2[0:1, c:c + 1] * hc)
        out_ref[...] = out


def kernel(dm1, edges_t, dm_t, edges_s, dm_s, edges_g, dm_g,
           W_t1, b_t1, W_t2, b_t2, W_s1, b_s1, W_s2, b_s2,
           W_g1, b_g1, W_g2, b_g2,
           fc1_W, fc1_b, fc2_W, fc2_b, cnn_W, cnn_b):
    counts = _sc_counts(edges_t.reshape(2 * _E), edges_s.reshape(2 * _E),
                        edges_g.reshape(2 * _E))

    dms = jnp.stack([dm_t, dm_s, dm_g])
    W1s = jnp.stack([W_t1, W_s1, W_g1])
    W2s = jnp.stack([W_t2, W_s2, W_g2])
    b1s = jnp.stack([b_t1, b_s1, b_g1]).reshape(3, 1, _FD)
    b2s = jnp.stack([b_t2, b_s2, b_g2]).reshape(3, 1, _FD)

    full = lambda shape: pl.BlockSpec(shape, lambda i: (0,) * len(shape))
    out = pl.pallas_call(
        _tc_body,
        grid=(3,),
        in_specs=[
            full((_N, _FD)),
            pl.BlockSpec((2 * _NB, _NB), lambda i: (i, 0)),
            pl.BlockSpec((1, _N, _N), lambda i: (i, 0, 0)),
            pl.BlockSpec((1, _FD, _FD), lambda i: (i, 0, 0)),
            pl.BlockSpec((1, 1, _FD), lambda i: (i, 0, 0)),
            pl.BlockSpec((1, _FD, _FD), lambda i: (i, 0, 0)),
            pl.BlockSpec((1, 1, _FD), lambda i: (i, 0, 0)),
            full((6, 30)), full((1, 30)), full((30, 6)), full((1, 6)),
            full((1, 6)), full((1, 1)),
        ],
        out_specs=pl.BlockSpec((_N, _FD), lambda i: (0, 0)),
        out_shape=jax.ShapeDtypeStruct((_N, _FD), jnp.float32),
        scratch_shapes=[pltpu.VMEM((6, _N, _FD), jnp.float32)],
        compiler_params=pltpu.CompilerParams(vmem_limit_bytes=100 * 1024 * 1024),
    )(dm1, counts, dms, W1s, b1s, W2s, b2s,
      fc1_W, fc1_b.reshape(1, 30), fc2_W, fc2_b.reshape(1, 6),
      cnn_W.reshape(1, 6), cnn_b.reshape(1, 1))
    return out


# speedup vs baseline: 79.9866x; 1.0020x over previous
import functools

import jax
import jax.numpy as jnp
from jax import lax
from jax.experimental import pallas as pl
from jax.experimental.pallas import tpu as pltpu
from jax.experimental.pallas import tpu_sc as plsc

_N = 591
_FD = 512
_E = 37824
_NB = 640
_BLK = _NB * _NB
_RPS = _NB // 16
_PW = _RPS * _NB
_CH = 1280
_NWIN = _CH // 128
_EAL = (_E // 128) * 128
_TC0 = _EAL - 128
_ZCH = _PW // 4


def _sc_counts(et, es, eg):
    mesh = plsc.VectorSubcoreMesh(core_axis_name="c", subcore_axis_name="s")

    @functools.partial(
        pl.kernel,
        out_type=jax.ShapeDtypeStruct((6 * _NB, _NB), jnp.float32),
        mesh=mesh,
        scratch_types=(
            [pltpu.VMEM((_CH,), jnp.int32)] * 6
            + [pltpu.VMEM((_NWIN, 128), jnp.int32)] * 3
            + [pltpu.VMEM((_NWIN, 128), jnp.float32)] * 3
            + [pltpu.VMEM((_ZCH,), jnp.float32)]
            + [pltpu.VMEM((_RPS, _NB), jnp.float32)] * 2
            + [pltpu.VMEM_SHARED((_BLK,), jnp.float32)] * 2
            + [pltpu.SemaphoreType.DMA] * 3
        ),
        compiler_params=pltpu.CompilerParams(needs_layout_passes=False),
    )
    def k(et_hbm, es_hbm, eg_hbm, out_hbm,
          es0, ed0, es1, ed1, es2, ed2,
          i0, i1, i2, v0, v1, v2,
          zb_v, p0, p1, blk0, blk1, sem_e, sem_s, sem_o):
        esrc_v = (es0, es1, es2)
        edst_v = (ed0, ed1, ed2)
        idx_v = (i0, i1, i2)
        val_v = (v0, v1, v2)
        cp_v = (p0, p1)
        blks = (blk0, blk1, blk0)
        cid = lax.axis_index("c")
        sid = lax.axis_index("s")
        wid = sid * 2 + cid
        lo = wid * _CH
        hi = jnp.minimum(lo + _CH, _E)
        c0 = jnp.minimum(lo, _E - _CH)

        ecopies = []
        for br, e_hbm in enumerate((et_hbm, es_hbm, eg_hbm)):
            ecopies.append(pltpu.async_copy(
                e_hbm.at[pl.ds(c0, _CH)], esrc_v[br], sem_e))
            ecopies.append(pltpu.async_copy(
                e_hbm.at[pl.ds(_E + c0, _CH)], edst_v[br], sem_e))

        @pl.loop(0, _ZCH, step=16)
        def _(i):
            zb_v[pl.ds(i, 16)] = jnp.zeros((16,), jnp.float32)

        zcopies = [
            pltpu.async_copy(
                zb_v, blk.at[pl.ds(sid * _PW + q * _ZCH, _ZCH)], sem_s)
            for blk in (blk0, blk1) for q in range(4)
        ]

        for c in ecopies:
            c.wait()

        for br in range(3):
            for w in range(_NWIN):
                @pl.loop(0, 128, step=16)
                def _(cc, br=br, w=w):
                    j = w * 128 + cc
                    s = esrc_v[br][pl.ds(j, 16)]
                    d = edst_v[br][pl.ds(j, 16)]
                    g = c0 + j + lax.iota(jnp.int32, 16)
                    m = (g >= lo) & (g < hi)
                    idx_v[br][w, pl.ds(cc, 16)] = s * _NB + d
                    val_v[br][w, pl.ds(cc, 16)] = m.astype(jnp.float32)

        for c in zcopies:
            c.wait()
        plsc.subcore_barrier()

        for br in (0, 1):
            scopies = []
            for w in range(_NWIN):
                scopies.append(pltpu.async_copy(
                    val_v[br].at[w], blks[br].at[idx_v[br].at[w]],
                    sem_s, add=True))
            for c in scopies:
                c.wait()
        plsc.subcore_barrier()

        hcopies = []
        for br in (0, 1):
            ocopies = [
                pltpu.async_copy(
                    blks[br].at[pl.ds(sid * _PW + r * _NB, _NB)],
                    cp_v[br].at[r], sem_o)
                for r in range(_RPS)
            ]
            for c in ocopies:
                c.wait()
            row0 = (br * 2 + cid) * _NB + sid * _RPS
            hcopies.append(pltpu.async_copy(
                cp_v[br], out_hbm.at[pl.ds(row0, _RPS), :], sem_e))
        rz = [
            pltpu.async_copy(
                zb_v, blk0.at[pl.ds(sid * _PW + q * _ZCH, _ZCH)], sem_s)
            for q in range(4)
        ]
        for c in rz:
            c.wait()
        plsc.subcore_barrier()

        scopies = []
        for w in range(_NWIN):
            scopies.append(pltpu.async_copy(
                val_v[2].at[w], blk0.at[idx_v[2].at[w]], sem_s, add=True))
        for c in scopies:
            c.wait()
        plsc.subcore_barrier()

        hcopies[0].wait()
        ocopies = [
            pltpu.async_copy(
                blk0.at[pl.ds(sid * _PW + r * _NB, _NB)],
                cp_v[0].at[r], sem_o)
            for r in range(_RPS)
        ]
        for c in ocopies:
            c.wait()
        row0 = (2 * 2 + cid) * _NB + sid * _RPS
        hlast = pltpu.async_copy(
            cp_v[0], out_hbm.at[pl.ds(row0, _RPS), :], sem_e)
        hcopies[1].wait()
        hlast.wait()

    return k(et, es, eg)


def _tc_body(x_ref, cnt_ref, dm_ref, w1_ref, b1_ref, w2_ref, b2_ref,
             f1w, f1b, f2w, f2b, cw, cb, out_ref, hs_ref):
    f32 = jnp.float32
    br = pl.program_id(0)
    x = x_ref[...]
    ones_col = jnp.ones((_N, 1), f32)
    tdims = (((0,), (0,)), ((), ()))

    Bc = (cnt_ref[pl.ds(0, _N), :_N]
          + cnt_ref[pl.ds(_NB, _N), :_N])
    M = Bc * dm_ref[0]
    deg_col = lax.dot_general(M, ones_col, tdims,
                              preferred_element_type=f32) + 1.0
    deg_row = jnp.sum(M, axis=0, keepdims=True) + 1.0
    dis_col = lax.rsqrt(deg_col)
    dis_row = lax.rsqrt(deg_row)
    Mn = M * dis_row * dis_col
    sl = dis_col * dis_col
    h = x
    for li, (W, b) in enumerate(((w1_ref, b1_ref), (w2_ref, b2_ref))):
        xw = jnp.dot(h, W[0], preferred_element_type=f32)
        agg = lax.dot_general(Mn, xw, tdims, preferred_element_type=f32)
        h = jax.nn.relu(agg + sl * xw + b[0])
        hs_ref[2 * br + li] = h

    @pl.when(br == 2)
    def _():
        inv = 1.0 / (_N * _FD)
        hs = [hs_ref[c] for c in range(6)]
        gap_row = jnp.concatenate(
            [jnp.sum(hc) * inv * jnp.ones((1, 1), f32) for hc in hs],
            axis=1)
        a1 = jax.nn.relu(jnp.dot(gap_row, f1w[...],
                                 preferred_element_type=f32) + f1b[...])
        a2 = jax.nn.sigmoid(jnp.dot(a1, f2w[...],
                                    preferred_element_type=f32) + f2b[...])
        cwv = cw[...]
        out = jnp.zeros((_N, _FD), f32) + cb[...]
        for c, hc in enumerate(hs):
            out = out + cwv[0:1, c:c + 1] * jax.nn.relu(a2[0:1, c:c + 1] * hc)
        out_ref[...] = out


def kernel(dm1, edges_t, dm_t, edges_s, dm_s, edges_g, dm_g,
           W_t1, b_t1, W_t2, b_t2, W_s1, b_s1, W_s2, b_s2,
           W_g1, b_g1, W_g2, b_g2,
           fc1_W, fc1_b, fc2_W, fc2_b, cnn_W, cnn_b):
    counts = _sc_counts(edges_t.reshape(2 * _E), edges_s.reshape(2 * _E),
                        edges_g.reshape(2 * _E))

    dms = jnp.stack([dm_t, dm_s, dm_g])
    W1s = jnp.stack([W_t1, W_s1, W_g1])
    W2s = jnp.stack([W_t2, W_s2, W_g2])
    b1s = jnp.stack([b_t1, b_s1, b_g1]).reshape(3, 1, _FD)
    b2s = jnp.stack([b_t2, b_s2, b_g2]).reshape(3, 1, _FD)

    full = lambda shape: pl.BlockSpec(shape, lambda i: (0,) * len(shape))
    out = pl.pallas_call(
        _tc_body,
        grid=(3,),
        in_specs=[
            full((_N, _FD)),
            pl.BlockSpec((2 * _NB, _NB), lambda i: (i, 0)),
            pl.BlockSpec((1, _N, _N), lambda i: (i, 0, 0)),
            pl.BlockSpec((1, _FD, _FD), lambda i: (i, 0, 0)),
            pl.BlockSpec((1, 1, _FD), lambda i: (i, 0, 0)),
            pl.BlockSpec((1, _FD, _FD), lambda i: (i, 0, 0)),
            pl.BlockSpec((1, 1, _FD), lambda i: (i, 0, 0)),
            full((6, 30)), full((1, 30)), full((30, 6)), full((1, 6)),
            full((1, 6)), full((1, 1)),
        ],
        out_specs=pl.BlockSpec((_N, _FD), lambda i: (0, 0)),
        out_shape=jax.ShapeDtypeStruct((_N, _FD), jnp.float32),
        scratch_shapes=[pltpu.VMEM((6, _N, _FD), jnp.float32)],
        compiler_params=pltpu.CompilerParams(vmem_limit_bytes=100 * 1024 * 1024),
    )(dm1, counts, dms, W1s, b1s, W2s, b2s,
      fc1_W, fc1_b.reshape(1, 30), fc2_W, fc2_b.reshape(1, 6),
      cnn_W.reshape(1, 6), cnn_b.reshape(1, 1))
    return out
